# Initial kernel scaffold; baseline (speedup 1.0000x reference)
import jax
import jax.numpy as jnp
from jax.experimental import pallas as pl
from jax.experimental.pallas import tpu as pltpu
from jax.experimental.pallas import tpu_sc as plsc

WINDOW = 128


def kernel(weights, indices):
    batch, hist_len = indices.shape
    _, embed_dim = weights.shape
    num_indices = batch * hist_len

    flat_idx = indices.reshape(1, num_indices)
    mesh = plsc.VectorSubcoreMesh(core_axis_name="core", subcore_axis_name="subcore")

    @pl.kernel(
        out_type=jax.ShapeDtypeStruct((num_indices, embed_dim), weights.dtype),
        mesh=mesh,
    )
    def gather_kernel(w_hbm, i_hbm, o_hbm):
        def body(i_vmem, o_vmem):
            pltpu.sync_copy(w_hbm.at[i_vmem.at[0]], o_vmem)

        pltpu.emit_pipeline(
            body,
            grid=(num_indices // WINDOW,),
            in_specs=[pl.BlockSpec((1, WINDOW), index_map=lambda i: (0, i))],
            out_specs=[pl.BlockSpec((WINDOW, embed_dim), index_map=lambda i: (i, 0))],
            core_axis_name=("core", "subcore"),
            dimension_semantics=(pltpu.PARALLEL,),
        )(i_hbm, o_hbm)

    out = gather_kernel(weights, flat_idx)
    return out.reshape(batch, hist_len, embed_dim)


# speedup vs baseline: 1.3995x; 1.3995x over previous
import functools

import jax
import jax.numpy as jnp
from jax import lax
from jax.experimental import pallas as pl
from jax.experimental.pallas import tpu as pltpu
from jax.experimental.pallas import tpu_sc as plsc

WINDOW = 128
NUM_CORES = 2
NUM_SUBCORES = 16
NUM_WORKERS = NUM_CORES * NUM_SUBCORES


def kernel(weights, indices):
    vocab, embed_dim = weights.shape
    batch, hist_len = indices.shape
    num_idx = batch * hist_len
    n_win = num_idx // WINDOW
    wpw = n_win // NUM_WORKERS

    idx2d = indices.reshape(n_win, WINDOW)
    mesh = plsc.VectorSubcoreMesh(core_axis_name="c", subcore_axis_name="s")

    @functools.partial(
        pl.kernel,
        mesh=mesh,
        compiler_params=pltpu.CompilerParams(use_tc_tiling_on_sc=False),
        out_type=jax.ShapeDtypeStruct((n_win, WINDOW, embed_dim), weights.dtype),
        scratch_types=[
            pltpu.VMEM((wpw, WINDOW), jnp.int32),
            pltpu.VMEM((WINDOW, embed_dim), jnp.float32),
            pltpu.SemaphoreType.DMA,
        ],
    )
    def gather_kernel(table_hbm, idx_hbm, out_hbm, idx_v, rows_v, sem):
        wid = lax.axis_index("s") * NUM_CORES + lax.axis_index("c")
        base = wid * wpw
        pltpu.sync_copy(idx_hbm.at[pl.ds(base, wpw)], idx_v)

        @pl.loop(0, wpw)
        def _(j):
            pltpu.async_copy(table_hbm.at[idx_v.at[j]], rows_v, sem).wait()
            pltpu.sync_copy(rows_v, out_hbm.at[base + j])

    out = gather_kernel(weights, idx2d)
    return out.reshape(batch, hist_len, embed_dim)


# speedup vs baseline: 1.4022x; 1.0019x over previous
import functools

import jax
import jax.numpy as jnp
from jax import lax
from jax.experimental import pallas as pl
from jax.experimental.pallas import tpu as pltpu
from jax.experimental.pallas import tpu_sc as plsc

WINDOW = 128
NUM_CORES = 2
NUM_SUBCORES = 16
NUM_WORKERS = NUM_CORES * NUM_SUBCORES


def kernel(weights, indices):
    vocab, embed_dim = weights.shape
    batch, hist_len = indices.shape
    num_idx = batch * hist_len
    n_win = num_idx // WINDOW
    wpw = n_win // NUM_WORKERS
    ipw = wpw * WINDOW

    flat_idx = indices.reshape(num_idx)
    mesh = plsc.VectorSubcoreMesh(core_axis_name="c", subcore_axis_name="s")

    @functools.partial(
        pl.kernel,
        mesh=mesh,
        compiler_params=pltpu.CompilerParams(use_tc_tiling_on_sc=False),
        out_type=jax.ShapeDtypeStruct((num_idx, embed_dim), weights.dtype),
        scratch_types=[
            pltpu.VMEM((ipw,), jnp.int32),
            pltpu.VMEM((WINDOW, embed_dim), jnp.float32),
            pltpu.SemaphoreType.DMA,
        ],
    )
    def gather_kernel(table_hbm, idx_hbm, out_hbm, idx_v, rows_v, sem):
        wid = lax.axis_index("s") * NUM_CORES + lax.axis_index("c")
        base = wid * ipw
        pltpu.sync_copy(idx_hbm.at[pl.ds(base, ipw)], idx_v)

        @pl.loop(0, wpw)
        def _(j):
            pltpu.async_copy(
                table_hbm.at[idx_v.at[pl.ds(j * WINDOW, WINDOW)]], rows_v, sem
            ).wait()
            pltpu.sync_copy(rows_v, out_hbm.at[pl.ds(base + j * WINDOW, WINDOW)])

    out = gather_kernel(weights, flat_idx)
    return out.reshape(batch, hist_len, embed_dim)


# speedup vs baseline: 1.7080x; 1.2181x over previous
import functools

import jax
import jax.numpy as jnp
from jax import lax
from jax.experimental import pallas as pl
from jax.experimental.pallas import tpu as pltpu
from jax.experimental.pallas import tpu_sc as plsc

WINDOW = 128
NUM_CORES = 2
NUM_SUBCORES = 16
NUM_WORKERS = NUM_CORES * NUM_SUBCORES

SPREAD_LANES = 4096


def _spread_body(wt_ref, out_ref):
    x = wt_ref[...]
    out_ref[:, 0:32] = jnp.swapaxes(x, 0, 1)


def _unpack_body(x_ref, o_ref):
    x = x_ref[...]
    y = jnp.swapaxes(x, 0, 1)
    o_ref[...] = y.reshape(o_ref.shape)


def kernel(weights, indices):
    vocab, embed_dim = weights.shape
    batch, hist_len = indices.shape
    num_idx = batch * hist_len
    n_win = num_idx // WINDOW
    wpw = n_win // NUM_WORKERS
    ipw = wpw * WINDOW

    flat_idx = indices.reshape(num_idx)

    wt = weights.T
    n_spread = (vocab + SPREAD_LANES - 1) // SPREAD_LANES
    w128 = pl.pallas_call(
        _spread_body,
        grid=(n_spread,),
        in_specs=[pl.BlockSpec((embed_dim, SPREAD_LANES), lambda i: (0, i))],
        out_specs=pl.BlockSpec((SPREAD_LANES, 128), lambda i: (i, 0)),
        out_shape=jax.ShapeDtypeStruct((vocab, 128), weights.dtype),
    )(wt)

    mesh = plsc.VectorSubcoreMesh(core_axis_name="c", subcore_axis_name="s")

    @functools.partial(
        pl.kernel,
        mesh=mesh,
        compiler_params=pltpu.CompilerParams(use_tc_tiling_on_sc=False),
        out_type=jax.ShapeDtypeStruct((num_idx, embed_dim), weights.dtype),
        scratch_types=[
            pltpu.VMEM((ipw,), jnp.int32),
            pltpu.VMEM((WINDOW, 128), jnp.float32),
            pltpu.SemaphoreType.DMA,
        ],
    )
    def gather_kernel(table_hbm, idx_hbm, out_hbm, idx_v, rows_v, sem):
        wid = lax.axis_index("s") * NUM_CORES + lax.axis_index("c")
        base = wid * ipw
        pltpu.sync_copy(idx_hbm.at[pl.ds(base, ipw)], idx_v)

        @pl.loop(0, wpw)
        def _(j):
            pltpu.async_copy(
                table_hbm.at[idx_v.at[pl.ds(j * WINDOW, WINDOW)]], rows_v, sem
            ).wait()
            pltpu.sync_copy(
                rows_v.at[:, pl.ds(0, embed_dim)],
                out_hbm.at[pl.ds(base + j * WINDOW, WINDOW)],
            )

    out = gather_kernel(w128, flat_idx)

    row = hist_len * embed_dim
    xb = out.reshape(batch, row)
    ot = pl.pallas_call(
        _unpack_body,
        grid=(batch // 128,),
        in_specs=[pl.BlockSpec((128, row), lambda i: (i, 0))],
        out_specs=pl.BlockSpec((hist_len, embed_dim, 128), lambda i: (0, 0, i)),
        out_shape=jax.ShapeDtypeStruct((hist_len, embed_dim, batch), weights.dtype),
    )(xb)
    return jnp.transpose(ot, (2, 0, 1))


# speedup vs baseline: 1.9659x; 1.1510x over previous
import functools

import jax
import jax.numpy as jnp
from jax import lax
from jax.experimental import pallas as pl
from jax.experimental.pallas import tpu as pltpu
from jax.experimental.pallas import tpu_sc as plsc

WINDOW = 128
NUM_CORES = 2
NUM_SUBCORES = 16
NUM_WORKERS = NUM_CORES * NUM_SUBCORES

SPREAD_LANES = 4096


def _spread_body(wt_ref, out_ref):
    x = wt_ref[...]
    out_ref[:, 0:32] = jnp.swapaxes(x, 0, 1)


def _unpack_body(x_ref, o_ref):
    x = x_ref[...]
    y = jnp.swapaxes(x, 0, 1)
    o_ref[...] = y.reshape(o_ref.shape)


def kernel(weights, indices):
    vocab, embed_dim = weights.shape
    batch, hist_len = indices.shape
    num_idx = batch * hist_len
    n_win = num_idx // WINDOW
    wpw = n_win // NUM_WORKERS
    ipw = wpw * WINDOW

    flat_idx = indices.reshape(num_idx) * 4

    wt = weights.T
    n_spread = (vocab + SPREAD_LANES - 1) // SPREAD_LANES
    w128 = pl.pallas_call(
        _spread_body,
        grid=(n_spread,),
        in_specs=[pl.BlockSpec((embed_dim, SPREAD_LANES), lambda i: (0, i))],
        out_specs=pl.BlockSpec((SPREAD_LANES, 128), lambda i: (i, 0)),
        out_shape=jax.ShapeDtypeStruct((vocab, 128), weights.dtype),
    )(wt)
    w4 = w128.reshape(4 * vocab, embed_dim)

    mesh = plsc.VectorSubcoreMesh(core_axis_name="c", subcore_axis_name="s")

    @functools.partial(
        pl.kernel,
        mesh=mesh,
        compiler_params=pltpu.CompilerParams(use_tc_tiling_on_sc=False),
        out_type=jax.ShapeDtypeStruct((num_idx, embed_dim), weights.dtype),
        scratch_types=[
            pltpu.VMEM((ipw,), jnp.int32),
            pltpu.VMEM((WINDOW, embed_dim), jnp.float32),
            pltpu.SemaphoreType.DMA,
        ],
    )
    def gather_kernel(table_hbm, idx_hbm, out_hbm, idx_v, rows_v, sem):
        wid = lax.axis_index("s") * NUM_CORES + lax.axis_index("c")
        base = wid * ipw
        pltpu.sync_copy(idx_hbm.at[pl.ds(base, ipw)], idx_v)

        @pl.loop(0, wpw)
        def _(j):
            pltpu.async_copy(
                table_hbm.at[idx_v.at[pl.ds(j * WINDOW, WINDOW)]], rows_v, sem
            ).wait()
            pltpu.sync_copy(rows_v, out_hbm.at[pl.ds(base + j * WINDOW, WINDOW)])

    out = gather_kernel(w4, flat_idx)

    row = hist_len * embed_dim
    xb = out.reshape(batch, row)
    ot = pl.pallas_call(
        _unpack_body,
        grid=(batch // 128,),
        in_specs=[pl.BlockSpec((128, row), lambda i: (i, 0))],
        out_specs=pl.BlockSpec((hist_len, embed_dim, 128), lambda i: (0, 0, i)),
        out_shape=jax.ShapeDtypeStruct((hist_len, embed_dim, batch), weights.dtype),
    )(xb)
    return jnp.transpose(ot, (2, 0, 1))


# speedup vs baseline: 1.9739x; 1.0040x over previous
import functools

import jax
import jax.numpy as jnp
from jax import lax
from jax.experimental import pallas as pl
from jax.experimental.pallas import tpu as pltpu
from jax.experimental.pallas import tpu_sc as plsc

WINDOW = 128
NUM_CORES = 2
NUM_SUBCORES = 16
NUM_WORKERS = NUM_CORES * NUM_SUBCORES

SPREAD_LANES = 4096


def _spread_body(wt_ref, out_ref):
    x = wt_ref[...]
    out_ref[:, 0:32] = jnp.swapaxes(x, 0, 1)


def _unpack_body(x_ref, o_ref):
    x = x_ref[...]
    y = jnp.swapaxes(x, 0, 1)
    o_ref[...] = y.reshape(o_ref.shape)


def kernel(weights, indices):
    vocab, embed_dim = weights.shape
    batch, hist_len = indices.shape
    num_idx = batch * hist_len
    n_win = num_idx // WINDOW
    wpw = n_win // NUM_WORKERS
    ipw = wpw * WINDOW

    flat_idx = indices.reshape(num_idx) * 4

    wt = weights.T
    n_spread = (vocab + SPREAD_LANES - 1) // SPREAD_LANES
    w128 = pl.pallas_call(
        _spread_body,
        grid=(n_spread,),
        in_specs=[pl.BlockSpec((embed_dim, SPREAD_LANES), lambda i: (0, i))],
        out_specs=pl.BlockSpec((SPREAD_LANES, 128), lambda i: (i, 0)),
        out_shape=jax.ShapeDtypeStruct((vocab, 128), weights.dtype),
        compiler_params=pltpu.CompilerParams(
            dimension_semantics=("parallel",)
        ),
    )(wt)
    w4 = w128.reshape(4 * vocab, embed_dim)

    mesh = plsc.VectorSubcoreMesh(core_axis_name="c", subcore_axis_name="s")

    @functools.partial(
        pl.kernel,
        mesh=mesh,
        compiler_params=pltpu.CompilerParams(use_tc_tiling_on_sc=False),
        out_type=jax.ShapeDtypeStruct((num_idx, embed_dim), weights.dtype),
        scratch_types=[
            pltpu.VMEM((ipw,), jnp.int32),
            pltpu.VMEM((WINDOW, embed_dim), jnp.float32),
            pltpu.SemaphoreType.DMA,
        ],
    )
    def gather_kernel(table_hbm, idx_hbm, out_hbm, idx_v, rows_v, sem):
        wid = lax.axis_index("s") * NUM_CORES + lax.axis_index("c")
        base = wid * ipw
        pltpu.sync_copy(idx_hbm.at[pl.ds(base, ipw)], idx_v)

        @pl.loop(0, wpw)
        def _(j):
            pltpu.async_copy(
                table_hbm.at[idx_v.at[pl.ds(j * WINDOW, WINDOW)]], rows_v, sem
            ).wait()
            pltpu.sync_copy(rows_v, out_hbm.at[pl.ds(base + j * WINDOW, WINDOW)])

    out = gather_kernel(w4, flat_idx)

    row = hist_len * embed_dim
    xb = out.reshape(batch, row)
    ot = pl.pallas_call(
        _unpack_body,
        grid=(batch // 128,),
        in_specs=[pl.BlockSpec((128, row), lambda i: (i, 0))],
        out_specs=pl.BlockSpec((hist_len, embed_dim, 128), lambda i: (0, 0, i)),
        out_shape=jax.ShapeDtypeStruct((hist_len, embed_dim, batch), weights.dtype),
        compiler_params=pltpu.CompilerParams(
            dimension_semantics=("parallel",)
        ),
    )(xb)
    return jnp.transpose(ot, (2, 0, 1))
